# Initial kernel scaffold; baseline (speedup 1.0000x reference)
import jax
import jax.numpy as jnp
from jax.experimental import pallas as pl


def kernel(data, angles, velo):
    raise NotImplementedError("write your pallas kernel here")


# speedup vs baseline: 2.2825x; 2.2825x over previous
import functools

import jax
import jax.numpy as jnp
from jax import lax
from jax.experimental import pallas as pl
from jax.experimental.pallas import tpu as pltpu
from jax.experimental.pallas import tpu_sc as plsc

BATCH = 16384
CH = 26
OUT = 64
NUM_STEPS = 3
LANES = 16
NUM_CORES = 2
NUM_SUBCORES = 16
NW = NUM_CORES * NUM_SUBCORES
ROWS_W = BATCH // NW
GROUPS = ROWS_W // LANES
MAGIC = float(2.0 ** 23)

_GATHER_DNUMS = lax.GatherDimensionNumbers(
    offset_dims=(), collapsed_slice_dims=(0,), start_index_map=(0,)
)


def _take16(table, idx):
    return lax.gather(
        table,
        idx[:, None],
        _GATHER_DNUMS,
        slice_sizes=(1,),
        mode=lax.GatherScatterMode.PROMISE_IN_BOUNDS,
    )


def _sc_body(data_hbm, vc_hbm, vs_hbm, out_hbm, data_v, out_v, vc_v, vs_v):
    wid = lax.axis_index("s") * NUM_CORES + lax.axis_index("c")
    base = wid * ROWS_W
    pltpu.sync_copy(data_hbm.at[:, pl.ds(base, ROWS_W)], data_v)
    pltpu.sync_copy(vc_hbm, vc_v)
    pltpu.sync_copy(vs_hbm, vs_v)

    tvc = vc_v[...]
    tvs = vs_v[...]

    def group(g, carry):
        rowbase = g * LANES
        acc = jnp.zeros((LANES,), jnp.float32)
        for c in range(CH):
            x = data_v[c, pl.ds(rowbase, LANES)]
            for _ in range(NUM_STEPS):
                t = 1.0 - 2.0 / (1.0 + jnp.exp(2.0 * x))
                r = jnp.minimum(((1.0 + t) * 8.0 + MAGIC) - MAGIC, 15.0)
                pos = r.astype(jnp.int32)
                val = _take16(tvc, pos) + t * _take16(tvs, pos)
                x = x + val / 3.0
            acc = acc + x
        for rloc in range(LANES):
            splat = _take16(acc, jnp.full((LANES,), rloc, jnp.int32))
            obase = (rowbase + rloc) * OUT
            for k in range(OUT // LANES):
                out_v[pl.ds(obase + k * LANES, LANES)] = splat
        return carry

    lax.fori_loop(0, GROUPS, group, 0)
    pltpu.sync_copy(out_v, out_hbm.at[pl.ds(wid * ROWS_W * OUT, ROWS_W * OUT)])


@jax.jit
def kernel(data, angles, velo):
    vc = velo * jnp.cos(angles)
    vs = velo * jnp.sin(angles)
    run = functools.partial(
        pl.kernel,
        out_type=jax.ShapeDtypeStruct((BATCH * OUT,), jnp.float32),
        mesh=plsc.VectorSubcoreMesh(core_axis_name="c", subcore_axis_name="s"),
        scratch_types=[
            pltpu.VMEM((CH, ROWS_W), jnp.float32),
            pltpu.VMEM((ROWS_W * OUT,), jnp.float32),
            pltpu.VMEM((LANES,), jnp.float32),
            pltpu.VMEM((LANES,), jnp.float32),
        ],
    )(_sc_body)
    out = run(data.T, vc, vs)
    return out.reshape(BATCH, OUT)


# speedup vs baseline: 2.9106x; 1.2752x over previous
import functools

import jax
import jax.numpy as jnp
from jax import lax
from jax.experimental import pallas as pl
from jax.experimental.pallas import tpu as pltpu
from jax.experimental.pallas import tpu_sc as plsc

BATCH = 16384
CH = 26
OUT = 64
NUM_STEPS = 3
LANES = 16
NUM_CORES = 2
NUM_SUBCORES = 16
NW = NUM_CORES * NUM_SUBCORES
ROWS_W = BATCH // NW
GROUPS = ROWS_W // LANES
MAGIC = float(2.0 ** 23)

_GATHER_DNUMS = lax.GatherDimensionNumbers(
    offset_dims=(), collapsed_slice_dims=(0,), start_index_map=(0,)
)


def _take16(table, idx):
    return lax.gather(
        table,
        idx[:, None],
        _GATHER_DNUMS,
        slice_sizes=(1,),
        mode=lax.GatherScatterMode.PROMISE_IN_BOUNDS,
    )


def _sc_body(data_hbm, vc_hbm, vs_hbm, out_hbm, data_v, out_v, vc_v, vs_v):
    wid = lax.axis_index("s") * NUM_CORES + lax.axis_index("c")
    base = wid * ROWS_W
    pltpu.sync_copy(data_hbm.at[:, pl.ds(base, ROWS_W)], data_v)
    pltpu.sync_copy(vc_hbm, vc_v)
    pltpu.sync_copy(vs_hbm, vs_v)

    tvc = vc_v[...]
    tvs = vs_v[...]

    def group(g, carry):
        rowbase = g * LANES
        acc = jnp.zeros((LANES,), jnp.float32)
        for c in range(CH):
            x = data_v[c, pl.ds(rowbase, LANES)]
            for _ in range(NUM_STEPS):
                u = 2.0 / (1.0 + jnp.exp(x + x))
                t = 1.0 - u
                r = jnp.minimum(((16.0 - 8.0 * u) + MAGIC) - MAGIC, 15.0)
                pos = r.astype(jnp.int32)
                x = (x + _take16(tvc, pos)) + t * _take16(tvs, pos)
            acc = acc + x
        for rloc in range(LANES):
            splat = _take16(acc, jnp.full((LANES,), rloc, jnp.int32))
            obase = (rowbase + rloc) * OUT
            for k in range(OUT // LANES):
                out_v[pl.ds(obase + k * LANES, LANES)] = splat
        return carry

    lax.fori_loop(0, GROUPS, group, 0)
    pltpu.sync_copy(out_v, out_hbm.at[pl.ds(wid * ROWS_W * OUT, ROWS_W * OUT)])


@jax.jit
def kernel(data, angles, velo):
    vc = velo * jnp.cos(angles) * (1.0 / NUM_STEPS)
    vs = velo * jnp.sin(angles) * (1.0 / NUM_STEPS)
    run = functools.partial(
        pl.kernel,
        out_type=jax.ShapeDtypeStruct((BATCH * OUT,), jnp.float32),
        mesh=plsc.VectorSubcoreMesh(core_axis_name="c", subcore_axis_name="s"),
        scratch_types=[
            pltpu.VMEM((CH, ROWS_W), jnp.float32),
            pltpu.VMEM((ROWS_W * OUT,), jnp.float32),
            pltpu.VMEM((LANES,), jnp.float32),
            pltpu.VMEM((LANES,), jnp.float32),
        ],
    )(_sc_body)
    out = run(data.T, vc, vs)
    return out.reshape(BATCH, OUT)


# speedup vs baseline: 2.9139x; 1.0011x over previous
import functools

import jax
import jax.numpy as jnp
from jax import lax
from jax.experimental import pallas as pl
from jax.experimental.pallas import tpu as pltpu
from jax.experimental.pallas import tpu_sc as plsc

BATCH = 16384
CH = 26
OUT = 64
NUM_STEPS = 3
LANES = 16
NUM_CORES = 2
NUM_SUBCORES = 16
NW = NUM_CORES * NUM_SUBCORES
ROWS_W = BATCH // NW
GROUPS = ROWS_W // LANES
MAGIC = float(2.0 ** 23)

_GATHER_DNUMS = lax.GatherDimensionNumbers(
    offset_dims=(), collapsed_slice_dims=(0,), start_index_map=(0,)
)


def _take16(table, idx):
    return lax.gather(
        table,
        idx[:, None],
        _GATHER_DNUMS,
        slice_sizes=(1,),
        mode=lax.GatherScatterMode.PROMISE_IN_BOUNDS,
    )


def _sc_body(data_hbm, vc_hbm, vs_hbm, out_hbm, data_v, out_v, vc_v, vs_v):
    wid = lax.axis_index("s") * NUM_CORES + lax.axis_index("c")
    base = wid * ROWS_W
    pltpu.sync_copy(data_hbm.at[:, pl.ds(base, ROWS_W)], data_v)
    pltpu.sync_copy(vc_hbm, vc_v)
    pltpu.sync_copy(vs_hbm, vs_v)

    tvc = vc_v[...]
    tvs = vs_v[...]

    def group(g, carry):
        rowbase = g * LANES
        acc = jnp.zeros((LANES,), jnp.float32)
        for c in range(CH):
            x = data_v[c, pl.ds(rowbase, LANES)]
            for _ in range(NUM_STEPS):
                u = 2.0 / (1.0 + jnp.exp(x + x))
                r = jnp.minimum(((16.0 - 8.0 * u) + MAGIC) - MAGIC, 15.0)
                pos = r.astype(jnp.int32)
                x = (x + _take16(tvc, pos)) - u * _take16(tvs, pos)
            acc = acc + x
        for rloc in range(LANES):
            splat = _take16(acc, jnp.full((LANES,), rloc, jnp.int32))
            obase = (rowbase + rloc) * OUT
            for k in range(OUT // LANES):
                out_v[pl.ds(obase + k * LANES, LANES)] = splat
        return carry

    lax.fori_loop(0, GROUPS, group, 0)
    pltpu.sync_copy(out_v, out_hbm.at[pl.ds(wid * ROWS_W * OUT, ROWS_W * OUT)])


@jax.jit
def kernel(data, angles, velo):
    vs = velo * jnp.sin(angles) * (1.0 / NUM_STEPS)
    vc = velo * jnp.cos(angles) * (1.0 / NUM_STEPS) + vs
    run = functools.partial(
        pl.kernel,
        out_type=jax.ShapeDtypeStruct((BATCH * OUT,), jnp.float32),
        mesh=plsc.VectorSubcoreMesh(core_axis_name="c", subcore_axis_name="s"),
        scratch_types=[
            pltpu.VMEM((CH, ROWS_W), jnp.float32),
            pltpu.VMEM((ROWS_W * OUT,), jnp.float32),
            pltpu.VMEM((LANES,), jnp.float32),
            pltpu.VMEM((LANES,), jnp.float32),
        ],
    )(_sc_body)
    out = run(data.T, vc, vs)
    return out.reshape(BATCH, OUT)
